# baseline (speedup 1.0000x reference)
import functools

import jax
import jax.numpy as jnp
from jax.experimental import pallas as pl
from jax.experimental.pallas import tpu as pltpu
from jax.experimental.pallas import tpu_sc as plsc


def _qkv_body(hid_ref, lnw_ref, wall_ref, cos_ref, sin_ref, qn_ref, kn_ref,
              out_ref, *, nq, nkv, eps):
    c = pl.program_id(0)
    x = hid_ref[...]
    var = jnp.mean(x * x, axis=-1, keepdims=True)
    hn = x * jax.lax.rsqrt(var + eps) * lnw_ref[...][None, :]
    y = jnp.dot(hn, wall_ref[0])

    @pl.when(c < nq + nkv)
    def _():
        nw = jnp.where(c < nq, qn_ref[...], kn_ref[...])
        v2 = jnp.mean(y * y, axis=-1, keepdims=True)
        yn = y * jax.lax.rsqrt(v2 + eps) * nw[None, :]
        half = yn.shape[-1] // 2
        x1 = yn[:, :half]
        x2 = yn[:, half:]
        rot = jnp.concatenate([-x2, x1], axis=1)
        out_ref[0] = yn * cos_ref[...] + rot * sin_ref[...]

    @pl.when(c >= nq + nkv)
    def _():
        out_ref[0] = y


def _attn_body(q_ref, k_ref, v_ref, o_ref, *, tq, sblk, scale):
    t = pl.program_id(1)
    q = q_ref[0] * scale
    h = q.shape[-1]
    nfull = (t * tq) // sblk

    def step(s, acc):
        k = k_ref[0, pl.ds(s * sblk, sblk), :]
        v = v_ref[0, pl.ds(s * sblk, sblk), :]
        ve = jnp.concatenate([v, jnp.ones((sblk, 1), jnp.float32)], axis=1)
        sc = jax.lax.dot_general(q, k, (((1,), (1,)), ((), ())))
        return acc + jnp.dot(jnp.exp(sc), ve)

    acc = jnp.zeros((tq, h + 1), jnp.float32)
    acc = jax.lax.fori_loop(0, nfull, step, acc)
    base = nfull * sblk
    k = k_ref[0, pl.ds(base, sblk), :]
    v = v_ref[0, pl.ds(base, sblk), :]
    ve = jnp.concatenate([v, jnp.ones((sblk, 1), jnp.float32)], axis=1)
    sc = jax.lax.dot_general(q, k, (((1,), (1,)), ((), ())))
    row = jax.lax.broadcasted_iota(jnp.int32, (tq, sblk), 0) + t * tq
    col = jax.lax.broadcasted_iota(jnp.int32, (tq, sblk), 1) + base
    p = jnp.where(col <= row, jnp.exp(sc), 0.0)
    acc = acc + jnp.dot(p, ve)
    o_ref[0] = acc[:, :h] / acc[:, h:]


def _post_body(a_ref, ow_ref, res_ref, lnw_ref, gw_ref, h_ref, hn_ref,
               lt_ref, *, nheads, eps):
    acc = jnp.dot(a_ref[0], ow_ref[0])
    for n in range(1, nheads):
        acc = acc + jnp.dot(a_ref[n], ow_ref[n])
    h = res_ref[...] + acc
    h_ref[...] = h
    var = jnp.mean(h * h, axis=-1, keepdims=True)
    hn = h * jax.lax.rsqrt(var + eps) * lnw_ref[...][None, :]
    hn_ref[...] = hn
    lt_ref[...] = jax.lax.dot_general(
        gw_ref[...], hn, (((0,), (1,)), ((), ())))


def _router_sc_body(lt_hbm, rw_hbm, lt_v, rw_v, *, ne, tpw):
    wid = jax.lax.axis_index("s") * 2 + jax.lax.axis_index("c")

    @pl.when(wid < 16)
    def _():
        base = wid * tpw
        pltpu.sync_copy(lt_hbm.at[:, pl.ds(base, tpw)], lt_v)
        lanes = 16
        for c in range(tpw // lanes):
            xs = [lt_v[e, pl.ds(c * lanes, lanes)] for e in range(ne)]
            m = xs[0]
            for e in range(1, ne):
                m = jnp.maximum(m, xs[e])
            ps = [jnp.exp(x - m) for x in xs]
            tot = ps[0]
            for e in range(1, ne):
                tot = tot + ps[e]
            probs = [p_ / tot for p_ in ps]
            m1 = probs[0]
            for e in range(1, ne):
                m1 = jnp.maximum(m1, probs[e])
            found = jnp.zeros((lanes,), jnp.float32)
            is1 = []
            for e in range(ne):
                hit = jnp.where(probs[e] == m1, 1.0, 0.0) * (1.0 - found)
                is1.append(hit)
                found = found + hit
            p2 = [probs[e] - 2.0 * is1[e] for e in range(ne)]
            m2 = p2[0]
            for e in range(1, ne):
                m2 = jnp.maximum(m2, p2[e])
            found = jnp.zeros((lanes,), jnp.float32)
            is2 = []
            for e in range(ne):
                hit = jnp.where(p2[e] == m2, 1.0, 0.0) * (1.0 - found)
                is2.append(hit)
                found = found + hit
            tot2 = m1 + m2
            w1 = m1 / tot2
            w2 = m2 / tot2
            for e in range(ne):
                rw_v[e, pl.ds(c * lanes, lanes)] = (
                    is1[e] * w1 + is2[e] * w2)
        pltpu.sync_copy(rw_v, rw_hbm.at[:, pl.ds(base, tpw)])


def _moe_body(hn_ref, rw_ref, res_ref, guw_ref, dw_ref, out_ref, *, f, ne):
    e = pl.program_id(0)
    hn = hn_ref[...]
    g = jnp.dot(hn, guw_ref[0, :, :f])
    u = jnp.dot(hn, guw_ref[0, :, f:])
    act = u * g / (1.0 + jnp.exp(-g))
    part = jnp.dot(act, dw_ref[0])
    w = jnp.transpose(rw_ref[0])
    part = part * w

    @pl.when(e == 0)
    def _():
        out_ref[...] = res_ref[...] + part

    @pl.when(e > 0)
    def _():
        out_ref[...] = out_ref[...] + part


def kernel(hidden_states, cos, sin, attention_mask, input_ln_w, post_ln_w,
           q_w, k_w, v_w, q_norm_w, k_norm_w, o_w, gate_w, gate_up_w, down_w):
    del attention_mask
    T, D = hidden_states.shape
    _, N, H = q_w.shape
    KV = k_w.shape[1]
    E = gate_w.shape[1]
    F = gate_up_w.shape[2] // 2
    C = N + 2 * KV
    eps = 1e-6

    wall = jnp.concatenate(
        [q_w.reshape(D, N * H), k_w.reshape(D, KV * H),
         v_w.reshape(D, KV * H)], axis=1)
    wall = wall.reshape(D, C, H).transpose(1, 0, 2)

    qkv = pl.pallas_call(
        functools.partial(_qkv_body, nq=N, nkv=KV, eps=eps),
        grid=(C,),
        in_specs=[
            pl.BlockSpec((T, D), lambda c: (0, 0)),
            pl.BlockSpec((D,), lambda c: (0,)),
            pl.BlockSpec((1, D, H), lambda c: (c, 0, 0)),
            pl.BlockSpec((T, H), lambda c: (0, 0)),
            pl.BlockSpec((T, H), lambda c: (0, 0)),
            pl.BlockSpec((H,), lambda c: (0,)),
            pl.BlockSpec((H,), lambda c: (0,)),
        ],
        out_specs=pl.BlockSpec((1, T, H), lambda c: (c, 0, 0)),
        out_shape=jax.ShapeDtypeStruct((C, T, H), jnp.float32),
    )(hidden_states, input_ln_w, wall, cos, sin, q_norm_w, k_norm_w)

    TQ = 512
    SBLK = 512
    g = N // KV
    attn = pl.pallas_call(
        functools.partial(_attn_body, tq=TQ, sblk=SBLK, scale=H ** -0.5),
        grid=(N, T // TQ),
        in_specs=[
            pl.BlockSpec((1, TQ, H), lambda n, t: (n, t, 0)),
            pl.BlockSpec((1, T, H), lambda n, t, g=g: (N + n // g, 0, 0)),
            pl.BlockSpec((1, T, H), lambda n, t, g=g: (N + KV + n // g, 0, 0)),
        ],
        out_specs=pl.BlockSpec((1, TQ, H), lambda n, t: (n, t, 0)),
        out_shape=jax.ShapeDtypeStruct((N, T, H), jnp.float32),
    )(qkv, qkv, qkv)

    TB = 256
    h, hn, lt = pl.pallas_call(
        functools.partial(_post_body, nheads=N, eps=eps),
        grid=(T // TB,),
        in_specs=[
            pl.BlockSpec((N, TB, H), lambda t: (0, t, 0)),
            pl.BlockSpec((N, H, D), lambda t: (0, 0, 0)),
            pl.BlockSpec((TB, D), lambda t: (t, 0)),
            pl.BlockSpec((D,), lambda t: (0,)),
            pl.BlockSpec((D, E), lambda t: (0, 0)),
        ],
        out_specs=[
            pl.BlockSpec((TB, D), lambda t: (t, 0)),
            pl.BlockSpec((TB, D), lambda t: (t, 0)),
            pl.BlockSpec((E, TB), lambda t: (0, t)),
        ],
        out_shape=[
            jax.ShapeDtypeStruct((T, D), jnp.float32),
            jax.ShapeDtypeStruct((T, D), jnp.float32),
            jax.ShapeDtypeStruct((E, T), jnp.float32),
        ],
    )(attn, o_w, hidden_states, post_ln_w, gate_w)

    TPW = T // 16
    rw = pl.kernel(
        functools.partial(_router_sc_body, ne=E, tpw=TPW),
        mesh=plsc.VectorSubcoreMesh(core_axis_name="c", subcore_axis_name="s"),
        out_type=jax.ShapeDtypeStruct((E, T), jnp.float32),
        scratch_types=[
            pltpu.VMEM((E, TPW), jnp.float32),
            pltpu.VMEM((E, TPW), jnp.float32),
        ],
    )(lt)

    out = pl.pallas_call(
        functools.partial(_moe_body, f=F, ne=E),
        grid=(E,),
        in_specs=[
            pl.BlockSpec((T, D), lambda e: (0, 0)),
            pl.BlockSpec((1, 1, T), lambda e: (e, 0, 0)),
            pl.BlockSpec((T, D), lambda e: (0, 0)),
            pl.BlockSpec((1, D, 2 * F), lambda e: (e, 0, 0)),
            pl.BlockSpec((1, F, D), lambda e: (e, 0, 0)),
        ],
        out_specs=pl.BlockSpec((T, D), lambda e: (0, 0)),
        out_shape=jax.ShapeDtypeStruct((T, D), jnp.float32),
    )(hn, rw.reshape(E, 1, T), h, gate_up_w, down_w)
    return out
